# baseline (speedup 1.0000x reference)
import functools

import jax
import jax.numpy as jnp
from jax import lax
from jax.experimental import pallas as pl
from jax.experimental.pallas import tpu as pltpu
from jax.experimental.pallas import tpu_sc as plsc

N = 10000
E = 320000
F32 = jnp.float32

BATCH = 128
EP = 327680
NROW = EP // BATCH
ROWS_PER_TILE = NROW // 16
ROWS_PER_WORKER = NROW // 32
NODES_PER_TILE = N // 16

_MESH = plsc.VectorSubcoreMesh(core_axis_name="c", subcore_axis_name="s")


@functools.partial(
    pl.kernel,
    mesh=_MESH,
    out_type=jax.ShapeDtypeStruct((2 * N,), F32),
    scratch_types=[
        pltpu.VMEM((ROWS_PER_WORKER // 5, BATCH), jnp.int32),
        pltpu.VMEM((ROWS_PER_WORKER // 5, BATCH), F32),
        pltpu.VMEM((2000,), F32),
        pltpu.VMEM_SHARED((N,), F32),
    ],
)
def _sc_deg(dst_hbm, w_hbm, z_hbm, out_hbm, dst_v, w_v, bounce_v, acc_sh):
    cid = lax.axis_index("c")
    sid = lax.axis_index("s")
    wid = sid * 2 + cid
    part = ROWS_PER_WORKER // 5

    @pl.when(sid == 0)
    def _():
        pltpu.sync_copy(z_hbm, acc_sh)

    plsc.subcore_barrier()

    def _chunk(ch, _):
        rbase = wid * ROWS_PER_WORKER + ch * part
        pltpu.sync_copy(dst_hbm.at[pl.ds(rbase, part)], dst_v)
        pltpu.sync_copy(w_hbm.at[pl.ds(rbase, part)], w_v)

        def _body(nb, _):
            pltpu.sync_copy(w_v.at[nb], acc_sh.at[dst_v.at[nb]], add=True)
            return 0

        lax.fori_loop(0, part, _body, 0)
        return 0

    lax.fori_loop(0, 5, _chunk, 0)

    plsc.subcore_barrier()

    @pl.when(sid == 0)
    def _():
        for r in range(5):
            pltpu.sync_copy(acc_sh.at[pl.ds(r * 2000, 2000)], bounce_v)
            pltpu.sync_copy(bounce_v,
                            out_hbm.at[pl.ds(cid * N + r * 2000, 2000)])


ECHUNK = 16
NOUTER = ROWS_PER_TILE // ECHUNK
DH = 128
NCH = DH // 16


@functools.partial(
    pl.kernel,
    mesh=_MESH,
    out_type=jax.ShapeDtypeStruct((2 * N, DH), F32),
    scratch_types=[
        pltpu.VMEM((ECHUNK, BATCH), jnp.int32),
        pltpu.VMEM((ECHUNK, BATCH), jnp.int32),
        pltpu.VMEM((ECHUNK, BATCH), F32),
        pltpu.VMEM((2, BATCH, DH), F32),
        pltpu.VMEM((16,), jnp.int32),
        pltpu.VMEM_SHARED((10240, DH), F32),
        pltpu.SemaphoreType.DMA,
        pltpu.SemaphoreType.DMA,
        pltpu.SemaphoreType.DMA,
        pltpu.SemaphoreType.DMA,
    ],
)
def _sc_conv(mode_hbm, src_hbm, dst_hbm, w_hbm, table_hbm, out_hbm,
             src_v, dst_v, w_v, rows_v, mode_v, acc_sh,
             sem0, sem1, ssem0, ssem1):
    cid = lax.axis_index("c")
    sid = lax.axis_index("s")
    pltpu.sync_copy(mode_hbm, mode_v)
    m = mode_v[...][0]
    tbase = (1 - m) * sid * ROWS_PER_TILE + m * (sid * 2 + cid) * ROWS_PER_WORKER
    nouter = NOUTER - m * (NOUTER // 2)
    off = cid * N
    sems = (sem0, sem1)
    ssems = (ssem0, ssem1)

    def _zloop(i, _):
        for k in range(NCH):
            rows_v[0, i, pl.ds(k * 16, 16)] = jnp.zeros((16,), F32)
        return 0

    lax.fori_loop(0, BATCH, _zloop, 0)
    for r in range(5):
        pltpu.sync_copy(rows_v.at[0],
                        acc_sh.at[pl.ds(sid * 640 + r * 128, 128)])
    plsc.subcore_barrier()

    def _chunk(ch, _):
        rbase = tbase + ch * ECHUNK
        pltpu.sync_copy(src_hbm.at[pl.ds(rbase, ECHUNK)], src_v)
        pltpu.sync_copy(dst_hbm.at[pl.ds(rbase, ECHUNK)], dst_v)
        pltpu.sync_copy(w_hbm.at[pl.ds(rbase, ECHUNK)], w_v)

        def _offs(i, _):
            for k in range(BATCH // 16):
                sl = pl.ds(k * 16, 16)
                src_v[i, sl] = src_v[i, sl] + off
            return 0

        lax.fori_loop(0, ECHUNK, _offs, 0)

        pltpu.async_copy(table_hbm.at[src_v.at[0]], rows_v.at[0], sem0)

        def _pair(p, _):
            for par in range(2):
                b = p * 2 + par
                other = 1 - par
                buf = rows_v.at[par]
                pltpu.make_async_copy(
                    table_hbm.at[src_v.at[b]], buf, sems[par]).wait()

                def _scale(g, _):
                    wchunk = w_v[b, pl.ds(g * 16, 16)]
                    for u in range(16):
                        s = wchunk[u]
                        e = g * 16 + u
                        for k in range(NCH):
                            sl = pl.ds(k * 16, 16)
                            rows_v[par, e, sl] = rows_v[par, e, sl] * s
                    return 0

                lax.fori_loop(0, BATCH // 16, _scale, 0)
                pltpu.async_copy(buf, acc_sh.at[dst_v.at[b]], ssems[par],
                                 add=True)

                @pl.when(b >= 1)
                def _():
                    pltpu.make_async_copy(
                        rows_v.at[other], acc_sh.at[dst_v.at[b - 1]],
                        ssems[other]).wait()

                @pl.when(b + 1 < ECHUNK)
                def _():
                    pltpu.async_copy(
                        table_hbm.at[src_v.at[b + 1]], rows_v.at[other],
                        sems[other])

            return 0

        lax.fori_loop(0, ECHUNK // 2, _pair, 0)
        pltpu.make_async_copy(
            rows_v.at[1], acc_sh.at[dst_v.at[ECHUNK - 1]], ssems[1]).wait()
        return 0

    lax.fori_loop(0, nouter, _chunk, 0)
    plsc.subcore_barrier()

    @pl.when(sid < 10)
    def _():
        pltpu.sync_copy(
            acc_sh.at[pl.ds(sid * 1000, 1000)],
            out_hbm.at[pl.ds(cid * N + sid * 1000, 1000)])


def _prep_body(deg_ref, x_ref, w1_ref, dinv_ref, h0p_ref):
    dinv = jax.lax.rsqrt(1.0 + deg_ref[0] + deg_ref[1])
    dinv_ref[...] = dinv
    h0 = jnp.dot(x_ref[...], w1_ref[...], preferred_element_type=F32)
    h0p = h0 * dinv
    h0p_ref[pl.ds(0, N), :] = h0p
    h0p_ref[pl.ds(N, N), :] = h0p


def _prep(deg2, xs, W1):
    return pl.pallas_call(
        _prep_body,
        out_shape=(
            jax.ShapeDtypeStruct((N, 1), F32),
            jax.ShapeDtypeStruct((2 * N, 128), F32),
        ),
    )(deg2, xs, W1)


def _mid_body(acc_ref, h0p_ref, dinv_ref, b1_ref, w2_ref, h1p_ref):
    dinv = dinv_ref[...]
    acc = acc_ref[pl.ds(0, N), :] + acc_ref[pl.ds(N, N), :]
    out1 = jnp.maximum(
        dinv * (acc + h0p_ref[pl.ds(0, N), :]) + b1_ref[...], 0.0)
    h1 = jnp.dot(out1, w2_ref[...], preferred_element_type=F32)
    h1p = h1 * dinv
    h1p_ref[pl.ds(0, N), :] = h1p[:, 0:128]
    h1p_ref[pl.ds(N, N), :] = h1p[:, 128:256]


def _mid(acc1, h0p, dinv, b1, W2):
    return pl.pallas_call(
        _mid_body,
        out_shape=jax.ShapeDtypeStruct((2 * N, 128), F32),
    )(acc1, h0p, dinv, b1, W2)


def _fin_body(acc_ref, h1p_ref, dinv_ref, b2_ref, out_ref):
    dinv = dinv_ref[...]
    out_ref[:, 0:128] = jnp.maximum(
        dinv * (acc_ref[pl.ds(0, N), :] + h1p_ref[pl.ds(0, N), :])
        + b2_ref[:, 0:128], 0.0)
    out_ref[:, 128:256] = jnp.maximum(
        dinv * (acc_ref[pl.ds(N, N), :] + h1p_ref[pl.ds(N, N), :])
        + b2_ref[:, 128:256], 0.0)


def _fin(acc2, h1p, dinv, b2):
    return pl.pallas_call(
        _fin_body,
        out_shape=jax.ShapeDtypeStruct((N, 256), F32),
    )(acc2, h1p, dinv, b2)


BK = 12800
KSTEPS = (N * 256) // BK


def _heads_body(flat_ref, wadv_ref, wv1_ref, badv_ref, bv1_ref,
                wv2t_ref, bv2_ref, wv3t_ref, bv3_ref,
                advp_ref, val_ref, acc_adv, acc_v1):
    g = pl.program_id(0)

    @pl.when(g == 0)
    def _():
        acc_adv[...] = jnp.zeros_like(acc_adv)
        acc_v1[...] = jnp.zeros_like(acc_v1)

    blk = flat_ref[...]
    acc_adv[...] += jnp.sum(wadv_ref[...] * blk, axis=1, keepdims=True)
    acc_v1[...] += jnp.sum(wv1_ref[...] * blk, axis=1, keepdims=True)

    @pl.when(g == KSTEPS - 1)
    def _():
        advp_ref[...] = jnp.maximum(acc_adv[...] + badv_ref[...], 0.0)
        v1 = jnp.maximum(acc_v1[...] + bv1_ref[...], 0.0)
        v2 = jnp.maximum(
            jnp.dot(wv2t_ref[...], v1, preferred_element_type=F32)
            + bv2_ref[...], 0.0)
        val_ref[...] = (
            jnp.dot(wv3t_ref[...], v2, preferred_element_type=F32)
            + bv3_ref[...])


def _heads(flat, WadvT, Wv1T, badv, bv1, Wv2T, bv2, Wv3T, bv3):
    return pl.pallas_call(
        _heads_body,
        grid=(KSTEPS,),
        in_specs=[
            pl.BlockSpec((1, BK), lambda g: (0, g)),
            pl.BlockSpec((15, BK), lambda g: (0, g)),
            pl.BlockSpec((64, BK), lambda g: (0, g)),
            pl.BlockSpec((15, 1), lambda g: (0, 0)),
            pl.BlockSpec((64, 1), lambda g: (0, 0)),
            pl.BlockSpec((64, 64), lambda g: (0, 0)),
            pl.BlockSpec((64, 1), lambda g: (0, 0)),
            pl.BlockSpec((1, 64), lambda g: (0, 0)),
            pl.BlockSpec((1, 1), lambda g: (0, 0)),
        ],
        out_specs=(
            pl.BlockSpec((15, 1), lambda g: (0, 0)),
            pl.BlockSpec((1, 1), lambda g: (0, 0)),
        ),
        out_shape=(
            jax.ShapeDtypeStruct((15, 1), F32),
            jax.ShapeDtypeStruct((1, 1), F32),
        ),
        scratch_shapes=[
            pltpu.VMEM((15, 1), F32),
            pltpu.VMEM((64, 1), F32),
        ],
    )(flat, WadvT, Wv1T, badv, bv1, Wv2T, bv2, Wv3T, bv3)


def kernel(x, edge_index, edge_weight, W1, b1, W2, b2,
           Wadv, badv, Wv1, bv1, Wv2, bv2, Wv3, bv3):
    B = x.shape[0]
    xs = x.reshape(B * N, 128)
    src = edge_index[0].astype(jnp.int32)
    dst = edge_index[1].astype(jnp.int32)

    pad = EP - E
    src2 = jnp.pad(src, (0, pad)).reshape(NROW, BATCH)
    dst2 = jnp.pad(dst, (0, pad)).reshape(NROW, BATCH)
    w2 = jnp.pad(edge_weight, (0, pad)).reshape(NROW, BATCH)

    zn = jnp.zeros((N,), F32)
    deg2 = _sc_deg(dst2, w2, zn).reshape(2, N, 1)
    dinv, h0p = _prep(deg2, xs, W1)
    m0 = jnp.zeros((16,), jnp.int32)
    m1 = jnp.ones((16,), jnp.int32)
    acc1 = _sc_conv(m1, src2, dst2, w2, h0p)
    h1p = _mid(acc1, h0p, dinv, b1.reshape(1, 128), W2)
    acc2 = _sc_conv(m0, src2, dst2, w2, h1p)
    out2 = _fin(acc2, h1p, dinv, b2.reshape(1, 256))

    flat = out2.reshape(1, N * 256)
    advp, val = _heads(flat, Wadv.T, Wv1.T, badv.reshape(15, 1),
                       bv1.reshape(64, 1), Wv2.T, bv2.reshape(64, 1),
                       Wv3.T, bv3.reshape(1, 1))
    adv3 = advp.reshape(B, 3, 5)
    return val[:, :, None] + adv3 - jnp.mean(adv3, axis=-1, keepdims=True)


# speedup vs baseline: 1.1055x; 1.1055x over previous
import functools

import jax
import jax.numpy as jnp
from jax import lax
from jax.experimental import pallas as pl
from jax.experimental.pallas import tpu as pltpu
from jax.experimental.pallas import tpu_sc as plsc

N = 10000
E = 320000
F32 = jnp.float32

BATCH = 128
EP = 327680
NROW = EP // BATCH
ROWS_PER_TILE = NROW // 16
ROWS_PER_WORKER = NROW // 32
NODES_PER_TILE = N // 16

_MESH = plsc.VectorSubcoreMesh(core_axis_name="c", subcore_axis_name="s")


@functools.partial(
    pl.kernel,
    mesh=_MESH,
    out_type=jax.ShapeDtypeStruct((2 * N,), F32),
    scratch_types=[
        pltpu.VMEM((ROWS_PER_WORKER // 5, BATCH), jnp.int32),
        pltpu.VMEM((ROWS_PER_WORKER // 5, BATCH), F32),
        pltpu.VMEM((2000,), F32),
        pltpu.VMEM_SHARED((N,), F32),
    ],
)
def _sc_deg(dst_hbm, w_hbm, z_hbm, out_hbm, dst_v, w_v, bounce_v, acc_sh):
    cid = lax.axis_index("c")
    sid = lax.axis_index("s")
    wid = sid * 2 + cid
    part = ROWS_PER_WORKER // 5

    @pl.when(sid == 0)
    def _():
        pltpu.sync_copy(z_hbm, acc_sh)

    plsc.subcore_barrier()

    def _chunk(ch, _):
        rbase = wid * ROWS_PER_WORKER + ch * part
        pltpu.sync_copy(dst_hbm.at[pl.ds(rbase, part)], dst_v)
        pltpu.sync_copy(w_hbm.at[pl.ds(rbase, part)], w_v)

        def _body(nb, _):
            pltpu.sync_copy(w_v.at[nb], acc_sh.at[dst_v.at[nb]], add=True)
            return 0

        lax.fori_loop(0, part, _body, 0)
        return 0

    lax.fori_loop(0, 5, _chunk, 0)

    plsc.subcore_barrier()

    @pl.when(sid == 0)
    def _():
        for r in range(5):
            pltpu.sync_copy(acc_sh.at[pl.ds(r * 2000, 2000)], bounce_v)
            pltpu.sync_copy(bounce_v,
                            out_hbm.at[pl.ds(cid * N + r * 2000, 2000)])


ECHUNK = 16
NOUTER = ROWS_PER_TILE // ECHUNK
DH = 128
NCH = DH // 16


@functools.partial(
    pl.kernel,
    mesh=_MESH,
    out_type=jax.ShapeDtypeStruct((2 * N, DH), F32),
    scratch_types=[
        pltpu.VMEM((ECHUNK, BATCH), jnp.int32),
        pltpu.VMEM((ECHUNK, BATCH), jnp.int32),
        pltpu.VMEM((ECHUNK, BATCH), F32),
        pltpu.VMEM((2, BATCH, DH), F32),
        pltpu.VMEM((16,), jnp.int32),
        pltpu.VMEM_SHARED((10240, DH), F32),
        pltpu.SemaphoreType.DMA,
        pltpu.SemaphoreType.DMA,
        pltpu.SemaphoreType.DMA,
        pltpu.SemaphoreType.DMA,
    ],
)
def _sc_conv(mode_hbm, src_hbm, dst_hbm, w_hbm, table_hbm, out_hbm,
             src_v, dst_v, w_v, rows_v, mode_v, acc_sh,
             sem0, sem1, ssem0, ssem1):
    cid = lax.axis_index("c")
    sid = lax.axis_index("s")
    pltpu.sync_copy(mode_hbm, mode_v)
    m = mode_v[...][0]
    tbase = (1 - m) * sid * ROWS_PER_TILE + m * (sid * 2 + cid) * ROWS_PER_WORKER
    nouter = NOUTER - m * (NOUTER // 2)
    off = cid * N
    sems = (sem0, sem1)
    ssems = (ssem0, ssem1)

    def _zloop(i, _):
        for k in range(NCH):
            rows_v[0, i, pl.ds(k * 16, 16)] = jnp.zeros((16,), F32)
        return 0

    lax.fori_loop(0, BATCH, _zloop, 0)
    for r in range(5):
        pltpu.sync_copy(rows_v.at[0],
                        acc_sh.at[pl.ds(sid * 640 + r * 128, 128)])
    plsc.subcore_barrier()

    def _chunk(ch, _):
        rbase = tbase + ch * ECHUNK
        pltpu.sync_copy(src_hbm.at[pl.ds(rbase, ECHUNK)], src_v)
        pltpu.sync_copy(dst_hbm.at[pl.ds(rbase, ECHUNK)], dst_v)
        pltpu.sync_copy(w_hbm.at[pl.ds(rbase, ECHUNK)], w_v)

        def _offs(i, _):
            for k in range(BATCH // 16):
                sl = pl.ds(k * 16, 16)
                src_v[i, sl] = src_v[i, sl] + off
            return 0

        lax.fori_loop(0, ECHUNK, _offs, 0)

        pltpu.async_copy(table_hbm.at[src_v.at[0]], rows_v.at[0], sem0)
        pltpu.async_copy(table_hbm.at[src_v.at[1]], rows_v.at[1], sem1)

        def _pair(p, _):
            for par in range(2):
                b = p * 2 + par
                buf = rows_v.at[par]
                pltpu.make_async_copy(
                    table_hbm.at[src_v.at[b]], buf, sems[par]).wait()

                def _scale(g, _):
                    wchunk = w_v[b, pl.ds(g * 16, 16)]
                    for u in range(16):
                        s = wchunk[u]
                        e = g * 16 + u
                        for k in range(NCH):
                            sl = pl.ds(k * 16, 16)
                            rows_v[par, e, sl] = rows_v[par, e, sl] * s
                    return 0

                lax.fori_loop(0, BATCH // 16, _scale, 0)
                pltpu.sync_copy(buf, acc_sh.at[dst_v.at[b]], add=True)

                @pl.when(b + 2 < ECHUNK)
                def _():
                    pltpu.async_copy(
                        table_hbm.at[src_v.at[b + 2]], buf, sems[par])

            return 0

        lax.fori_loop(0, ECHUNK // 2, _pair, 0)
        return 0

    lax.fori_loop(0, nouter, _chunk, 0)
    plsc.subcore_barrier()

    @pl.when(sid < 10)
    def _():
        pltpu.sync_copy(
            acc_sh.at[pl.ds(sid * 1000, 1000)],
            out_hbm.at[pl.ds(cid * N + sid * 1000, 1000)])


def _prep_body(deg_ref, x_ref, w1_ref, dinv_ref, h0p_ref):
    dinv = jax.lax.rsqrt(1.0 + deg_ref[0] + deg_ref[1])
    dinv_ref[...] = dinv
    h0 = jnp.dot(x_ref[...], w1_ref[...], preferred_element_type=F32)
    h0p = h0 * dinv
    h0p_ref[pl.ds(0, N), :] = h0p
    h0p_ref[pl.ds(N, N), :] = h0p


def _prep(deg2, xs, W1):
    return pl.pallas_call(
        _prep_body,
        out_shape=(
            jax.ShapeDtypeStruct((N, 1), F32),
            jax.ShapeDtypeStruct((2 * N, 128), F32),
        ),
    )(deg2, xs, W1)


def _mid_body(acc_ref, h0p_ref, dinv_ref, b1_ref, w2_ref, h1p_ref):
    dinv = dinv_ref[...]
    acc = acc_ref[pl.ds(0, N), :] + acc_ref[pl.ds(N, N), :]
    out1 = jnp.maximum(
        dinv * (acc + h0p_ref[pl.ds(0, N), :]) + b1_ref[...], 0.0)
    h1 = jnp.dot(out1, w2_ref[...], preferred_element_type=F32)
    h1p = h1 * dinv
    h1p_ref[pl.ds(0, N), :] = h1p[:, 0:128]
    h1p_ref[pl.ds(N, N), :] = h1p[:, 128:256]


def _mid(acc1, h0p, dinv, b1, W2):
    return pl.pallas_call(
        _mid_body,
        out_shape=jax.ShapeDtypeStruct((2 * N, 128), F32),
    )(acc1, h0p, dinv, b1, W2)


def _fin_body(acc_ref, h1p_ref, dinv_ref, b2_ref, out_ref):
    dinv = dinv_ref[...]
    out_ref[:, 0:128] = jnp.maximum(
        dinv * (acc_ref[pl.ds(0, N), :] + h1p_ref[pl.ds(0, N), :])
        + b2_ref[:, 0:128], 0.0)
    out_ref[:, 128:256] = jnp.maximum(
        dinv * (acc_ref[pl.ds(N, N), :] + h1p_ref[pl.ds(N, N), :])
        + b2_ref[:, 128:256], 0.0)


def _fin(acc2, h1p, dinv, b2):
    return pl.pallas_call(
        _fin_body,
        out_shape=jax.ShapeDtypeStruct((N, 256), F32),
    )(acc2, h1p, dinv, b2)


BK = 12800
KSTEPS = (N * 256) // BK


def _heads_body(flat_ref, wadv_ref, wv1_ref, badv_ref, bv1_ref,
                wv2t_ref, bv2_ref, wv3t_ref, bv3_ref,
                advp_ref, val_ref, acc_adv, acc_v1):
    g = pl.program_id(0)

    @pl.when(g == 0)
    def _():
        acc_adv[...] = jnp.zeros_like(acc_adv)
        acc_v1[...] = jnp.zeros_like(acc_v1)

    blk = flat_ref[...]
    acc_adv[...] += jnp.sum(wadv_ref[...] * blk, axis=1, keepdims=True)
    acc_v1[...] += jnp.sum(wv1_ref[...] * blk, axis=1, keepdims=True)

    @pl.when(g == KSTEPS - 1)
    def _():
        advp_ref[...] = jnp.maximum(acc_adv[...] + badv_ref[...], 0.0)
        v1 = jnp.maximum(acc_v1[...] + bv1_ref[...], 0.0)
        v2 = jnp.maximum(
            jnp.dot(wv2t_ref[...], v1, preferred_element_type=F32)
            + bv2_ref[...], 0.0)
        val_ref[...] = (
            jnp.dot(wv3t_ref[...], v2, preferred_element_type=F32)
            + bv3_ref[...])


def _heads(flat, WadvT, Wv1T, badv, bv1, Wv2T, bv2, Wv3T, bv3):
    return pl.pallas_call(
        _heads_body,
        grid=(KSTEPS,),
        in_specs=[
            pl.BlockSpec((1, BK), lambda g: (0, g)),
            pl.BlockSpec((15, BK), lambda g: (0, g)),
            pl.BlockSpec((64, BK), lambda g: (0, g)),
            pl.BlockSpec((15, 1), lambda g: (0, 0)),
            pl.BlockSpec((64, 1), lambda g: (0, 0)),
            pl.BlockSpec((64, 64), lambda g: (0, 0)),
            pl.BlockSpec((64, 1), lambda g: (0, 0)),
            pl.BlockSpec((1, 64), lambda g: (0, 0)),
            pl.BlockSpec((1, 1), lambda g: (0, 0)),
        ],
        out_specs=(
            pl.BlockSpec((15, 1), lambda g: (0, 0)),
            pl.BlockSpec((1, 1), lambda g: (0, 0)),
        ),
        out_shape=(
            jax.ShapeDtypeStruct((15, 1), F32),
            jax.ShapeDtypeStruct((1, 1), F32),
        ),
        scratch_shapes=[
            pltpu.VMEM((15, 1), F32),
            pltpu.VMEM((64, 1), F32),
        ],
    )(flat, WadvT, Wv1T, badv, bv1, Wv2T, bv2, Wv3T, bv3)


def kernel(x, edge_index, edge_weight, W1, b1, W2, b2,
           Wadv, badv, Wv1, bv1, Wv2, bv2, Wv3, bv3):
    B = x.shape[0]
    xs = x.reshape(B * N, 128)
    src = edge_index[0].astype(jnp.int32)
    dst = edge_index[1].astype(jnp.int32)

    pad = EP - E
    src2 = jnp.pad(src, (0, pad)).reshape(NROW, BATCH)
    dst2 = jnp.pad(dst, (0, pad)).reshape(NROW, BATCH)
    w2 = jnp.pad(edge_weight, (0, pad)).reshape(NROW, BATCH)

    zn = jnp.zeros((N,), F32)
    deg2 = _sc_deg(dst2, w2, zn).reshape(2, N, 1)
    dinv, h0p = _prep(deg2, xs, W1)
    m0 = jnp.zeros((16,), jnp.int32)
    m1 = jnp.ones((16,), jnp.int32)
    acc1 = _sc_conv(m1, src2, dst2, w2, h0p)
    h1p = _mid(acc1, h0p, dinv, b1.reshape(1, 128), W2)
    acc2 = _sc_conv(m0, src2, dst2, w2, h1p)
    out2 = _fin(acc2, h1p, dinv, b2.reshape(1, 256))

    flat = out2.reshape(1, N * 256)
    advp, val = _heads(flat, Wadv.T, Wv1.T, badv.reshape(15, 1),
                       bv1.reshape(64, 1), Wv2.T, bv2.reshape(64, 1),
                       Wv3.T, bv3.reshape(1, 1))
    adv3 = advp.reshape(B, 3, 5)
    return val[:, :, None] + adv3 - jnp.mean(adv3, axis=-1, keepdims=True)


# speedup vs baseline: 1.1060x; 1.0004x over previous
import functools

import jax
import jax.numpy as jnp
from jax import lax
from jax.experimental import pallas as pl
from jax.experimental.pallas import tpu as pltpu
from jax.experimental.pallas import tpu_sc as plsc

N = 10000
E = 320000
F32 = jnp.float32

BATCH = 128
EP = 327680
NROW = EP // BATCH
ROWS_PER_TILE = NROW // 16
ROWS_PER_WORKER = NROW // 32
NODES_PER_TILE = N // 16

_MESH = plsc.VectorSubcoreMesh(core_axis_name="c", subcore_axis_name="s")


@functools.partial(
    pl.kernel,
    mesh=_MESH,
    out_type=jax.ShapeDtypeStruct((2 * N,), F32),
    scratch_types=[
        pltpu.VMEM((ROWS_PER_WORKER // 5, BATCH), jnp.int32),
        pltpu.VMEM((ROWS_PER_WORKER // 5, BATCH), F32),
        pltpu.VMEM((2000,), F32),
        pltpu.VMEM_SHARED((N,), F32),
    ],
)
def _sc_deg(dst_hbm, w_hbm, z_hbm, out_hbm, dst_v, w_v, bounce_v, acc_sh):
    cid = lax.axis_index("c")
    sid = lax.axis_index("s")
    wid = sid * 2 + cid
    part = ROWS_PER_WORKER // 5

    @pl.when(sid == 0)
    def _():
        pltpu.sync_copy(z_hbm, acc_sh)

    plsc.subcore_barrier()

    def _chunk(ch, _):
        rbase = wid * ROWS_PER_WORKER + ch * part
        pltpu.sync_copy(dst_hbm.at[pl.ds(rbase, part)], dst_v)
        pltpu.sync_copy(w_hbm.at[pl.ds(rbase, part)], w_v)

        def _body(nb, _):
            pltpu.sync_copy(w_v.at[nb], acc_sh.at[dst_v.at[nb]], add=True)
            return 0

        lax.fori_loop(0, part, _body, 0)
        return 0

    lax.fori_loop(0, 5, _chunk, 0)

    plsc.subcore_barrier()

    @pl.when(sid == 0)
    def _():
        for r in range(5):
            pltpu.sync_copy(acc_sh.at[pl.ds(r * 2000, 2000)], bounce_v)
            pltpu.sync_copy(bounce_v,
                            out_hbm.at[pl.ds(cid * N + r * 2000, 2000)])


ECHUNK = 16
NOUTER = ROWS_PER_TILE // ECHUNK
DH = 128
NCH = DH // 16


@functools.partial(
    pl.kernel,
    mesh=_MESH,
    out_type=jax.ShapeDtypeStruct((2 * N, DH), F32),
    scratch_types=[
        pltpu.VMEM((ECHUNK, BATCH), jnp.int32),
        pltpu.VMEM((ECHUNK, BATCH), jnp.int32),
        pltpu.VMEM((ECHUNK, BATCH), F32),
        pltpu.VMEM((2, BATCH, DH), F32),
        pltpu.VMEM((16,), jnp.int32),
        pltpu.VMEM_SHARED((10240, DH), F32),
        pltpu.SemaphoreType.DMA,
        pltpu.SemaphoreType.DMA,
    ],
)
def _sc_conv(mode_hbm, src_hbm, dst_hbm, w_hbm, table_hbm, out_hbm,
             src_v, dst_v, w_v, rows_v, mode_v, acc_sh, sem0, sem1):
    cid = lax.axis_index("c")
    sid = lax.axis_index("s")
    pltpu.sync_copy(mode_hbm, mode_v)
    m = mode_v[...][0]
    tbase = (1 - m) * sid * ROWS_PER_TILE + m * (sid * 2 + cid) * ROWS_PER_WORKER
    nouter = NOUTER - m * (NOUTER // 2)
    off = cid * N
    sems = (sem0, sem1)

    def _zloop(i, _):
        for k in range(NCH):
            rows_v[0, i, pl.ds(k * 16, 16)] = jnp.zeros((16,), F32)
        return 0

    lax.fori_loop(0, BATCH, _zloop, 0)
    for r in range(5):
        pltpu.sync_copy(rows_v.at[0],
                        acc_sh.at[pl.ds(sid * 640 + r * 128, 128)])
    plsc.subcore_barrier()

    def _chunk(ch, _):
        rbase = tbase + ch * ECHUNK
        pltpu.sync_copy(src_hbm.at[pl.ds(rbase, ECHUNK)], src_v)
        pltpu.sync_copy(dst_hbm.at[pl.ds(rbase, ECHUNK)], dst_v)
        pltpu.sync_copy(w_hbm.at[pl.ds(rbase, ECHUNK)], w_v)

        def _offs(i, _):
            for k in range(BATCH // 16):
                sl = pl.ds(k * 16, 16)
                src_v[i, sl] = src_v[i, sl] + off
            return 0

        lax.fori_loop(0, ECHUNK, _offs, 0)

        pltpu.async_copy(table_hbm.at[src_v.at[0]], rows_v.at[0], sem0)
        pltpu.async_copy(table_hbm.at[src_v.at[1]], rows_v.at[1], sem1)

        def _pair(p, _):
            for par in range(2):
                b = p * 2 + par
                buf = rows_v.at[par]
                pltpu.make_async_copy(
                    table_hbm.at[src_v.at[b]], buf, sems[par]).wait()

                def _scale(g, _):
                    wchunk = w_v[b, pl.ds(g * 16, 16)]
                    for u in range(16):
                        s = wchunk[u]
                        e = g * 16 + u
                        for k in range(NCH):
                            sl = pl.ds(k * 16, 16)
                            rows_v[par, e, sl] = rows_v[par, e, sl] * s
                    return 0

                lax.fori_loop(0, BATCH // 16, _scale, 0)
                pltpu.sync_copy(buf, acc_sh.at[dst_v.at[b]], add=True)

                @pl.when(b + 2 < ECHUNK)
                def _():
                    pltpu.async_copy(
                        table_hbm.at[src_v.at[b + 2]], buf, sems[par])

            return 0

        lax.fori_loop(0, ECHUNK // 2, _pair, 0)
        return 0

    lax.fori_loop(0, nouter, _chunk, 0)
    plsc.subcore_barrier()

    @pl.when(sid < 10)
    def _():
        pltpu.sync_copy(
            acc_sh.at[pl.ds(sid * 1000, 1000)],
            out_hbm.at[pl.ds(cid * N + sid * 1000, 1000)])


def _prep_body(deg_ref, x_ref, w1_ref, dinv_ref, h0p_ref):
    dinv = jax.lax.rsqrt(1.0 + deg_ref[0] + deg_ref[1])
    dinv_ref[...] = dinv
    h0 = jnp.dot(x_ref[...], w1_ref[...], preferred_element_type=F32)
    h0p = h0 * dinv
    h0p_ref[pl.ds(0, N), :] = h0p
    h0p_ref[pl.ds(N, N), :] = h0p


def _prep(deg2, xs, W1):
    return pl.pallas_call(
        _prep_body,
        out_shape=(
            jax.ShapeDtypeStruct((N, 1), F32),
            jax.ShapeDtypeStruct((2 * N, 128), F32),
        ),
    )(deg2, xs, W1)


def _mid_body(acc_ref, h0p_ref, dinv_ref, b1_ref, w2_ref, h1p_ref):
    dinv = dinv_ref[...]
    acc = acc_ref[pl.ds(0, N), :] + acc_ref[pl.ds(N, N), :]
    out1 = jnp.maximum(
        dinv * (acc + h0p_ref[pl.ds(0, N), :]) + b1_ref[...], 0.0)
    h1 = jnp.dot(out1, w2_ref[...], preferred_element_type=F32)
    h1p = h1 * dinv
    h1p_ref[pl.ds(0, N), :] = h1p[:, 0:128]
    h1p_ref[pl.ds(N, N), :] = h1p[:, 128:256]


def _mid(acc1, h0p, dinv, b1, W2):
    return pl.pallas_call(
        _mid_body,
        out_shape=jax.ShapeDtypeStruct((2 * N, 128), F32),
    )(acc1, h0p, dinv, b1, W2)


def _fin_body(acc_ref, h1p_ref, dinv_ref, b2_ref, out_ref):
    dinv = dinv_ref[...]
    out_ref[:, 0:128] = jnp.maximum(
        dinv * (acc_ref[pl.ds(0, N), :] + h1p_ref[pl.ds(0, N), :])
        + b2_ref[:, 0:128], 0.0)
    out_ref[:, 128:256] = jnp.maximum(
        dinv * (acc_ref[pl.ds(N, N), :] + h1p_ref[pl.ds(N, N), :])
        + b2_ref[:, 128:256], 0.0)


def _fin(acc2, h1p, dinv, b2):
    return pl.pallas_call(
        _fin_body,
        out_shape=jax.ShapeDtypeStruct((N, 256), F32),
    )(acc2, h1p, dinv, b2)


BK = 12800
KSTEPS = (N * 256) // BK


def _heads_body(flat_ref, wadv_ref, wv1_ref, badv_ref, bv1_ref,
                wv2t_ref, bv2_ref, wv3t_ref, bv3_ref,
                advp_ref, val_ref, acc_adv, acc_v1):
    g = pl.program_id(0)

    @pl.when(g == 0)
    def _():
        acc_adv[...] = jnp.zeros_like(acc_adv)
        acc_v1[...] = jnp.zeros_like(acc_v1)

    blk = flat_ref[...]
    acc_adv[...] += jnp.sum(wadv_ref[...] * blk, axis=1, keepdims=True)
    acc_v1[...] += jnp.sum(wv1_ref[...] * blk, axis=1, keepdims=True)

    @pl.when(g == KSTEPS - 1)
    def _():
        advp_ref[...] = jnp.maximum(acc_adv[...] + badv_ref[...], 0.0)
        v1 = jnp.maximum(acc_v1[...] + bv1_ref[...], 0.0)
        v2 = jnp.maximum(
            jnp.dot(wv2t_ref[...], v1, preferred_element_type=F32)
            + bv2_ref[...], 0.0)
        val_ref[...] = (
            jnp.dot(wv3t_ref[...], v2, preferred_element_type=F32)
            + bv3_ref[...])


def _heads(flat, WadvT, Wv1T, badv, bv1, Wv2T, bv2, Wv3T, bv3):
    return pl.pallas_call(
        _heads_body,
        grid=(KSTEPS,),
        in_specs=[
            pl.BlockSpec((1, BK), lambda g: (0, g)),
            pl.BlockSpec((15, BK), lambda g: (0, g)),
            pl.BlockSpec((64, BK), lambda g: (0, g)),
            pl.BlockSpec((15, 1), lambda g: (0, 0)),
            pl.BlockSpec((64, 1), lambda g: (0, 0)),
            pl.BlockSpec((64, 64), lambda g: (0, 0)),
            pl.BlockSpec((64, 1), lambda g: (0, 0)),
            pl.BlockSpec((1, 64), lambda g: (0, 0)),
            pl.BlockSpec((1, 1), lambda g: (0, 0)),
        ],
        out_specs=(
            pl.BlockSpec((15, 1), lambda g: (0, 0)),
            pl.BlockSpec((1, 1), lambda g: (0, 0)),
        ),
        out_shape=(
            jax.ShapeDtypeStruct((15, 1), F32),
            jax.ShapeDtypeStruct((1, 1), F32),
        ),
        scratch_shapes=[
            pltpu.VMEM((15, 1), F32),
            pltpu.VMEM((64, 1), F32),
        ],
    )(flat, WadvT, Wv1T, badv, bv1, Wv2T, bv2, Wv3T, bv3)


def kernel(x, edge_index, edge_weight, W1, b1, W2, b2,
           Wadv, badv, Wv1, bv1, Wv2, bv2, Wv3, bv3):
    B = x.shape[0]
    xs = x.reshape(B * N, 128)
    src = edge_index[0].astype(jnp.int32)
    dst = edge_index[1].astype(jnp.int32)

    pad = EP - E
    src2 = jnp.pad(src, (0, pad)).reshape(NROW, BATCH)
    dst2 = jnp.pad(dst, (0, pad)).reshape(NROW, BATCH)
    w2 = jnp.pad(edge_weight, (0, pad)).reshape(NROW, BATCH)

    zn = jnp.zeros((N,), F32)
    deg2 = _sc_deg(dst2, w2, zn).reshape(2, N, 1)
    dinv, h0p = _prep(deg2, xs, W1)
    m0 = jnp.zeros((16,), jnp.int32)
    m1 = jnp.ones((16,), jnp.int32)
    acc1 = _sc_conv(m1, src2, dst2, w2, h0p)
    h1p = _mid(acc1, h0p, dinv, b1.reshape(1, 128), W2)
    acc2 = _sc_conv(m0, src2, dst2, w2, h1p)
    out2 = _fin(acc2, h1p, dinv, b2.reshape(1, 256))

    flat = out2.reshape(1, N * 256)
    advp, val = _heads(flat, Wadv.T, Wv1.T, badv.reshape(15, 1),
                       bv1.reshape(64, 1), Wv2.T, bv2.reshape(64, 1),
                       Wv3.T, bv3.reshape(1, 1))
    adv3 = advp.reshape(B, 3, 5)
    return val[:, :, None] + adv3 - jnp.mean(adv3, axis=-1, keepdims=True)
